# baseline (speedup 1.0000x reference)
import jax
import jax.numpy as jnp
from jax import lax
from jax.experimental import pallas as pl
from jax.experimental.pallas import tpu as pltpu
from jax.experimental.pallas import tpu_sc as plsc

N = 10000
NNZ = 160000
B = 8
LATENT = 32
UNITS = 64
NUM_MAT = 5

NC = 2
NS = 16
FB = 64
NBLK = 4
EPT = NNZ // NS
CH = 80
NCHUNK = EPT // CH
RPT = 624
RPT_LAST = N - 15 * RPT


def _spmm_body(*refs):
    xs = refs[:NBLK]
    rows, cols, vals = refs[NBLK:NBLK + 3]
    outs = refs[NBLK + 3:NBLK + 3 + NBLK]
    colv, rowv, valv, gath, zbuf, acc, sem = refs[NBLK + 3 + NBLK:]
    c = lax.axis_index("c")
    s = lax.axis_index("s")
    zeros16 = jnp.zeros((16,), jnp.float32)

    def zb(i, carry):
        for j in range(FB // 16):
            zbuf[i, pl.ds(j * 16, 16)] = zeros16
        return carry

    lax.fori_loop(0, RPT_LAST, zb, 0)
    is_last = s == NS - 1

    for p in range(NBLK // 2):
        @pl.when(jnp.logical_not(is_last))
        def _():
            pltpu.sync_copy(zbuf.at[pl.ds(0, RPT)], acc.at[pl.ds(s * RPT, RPT)])

        @pl.when(is_last)
        def _():
            pltpu.sync_copy(zbuf, acc.at[pl.ds(15 * RPT, RPT_LAST)])

        plsc.subcore_barrier()
        for cc in range(NC):
            blk = p * 2 + cc

            @pl.when(c == cc)
            def _(blk=blk):
                x = xs[blk]

                def chunk(i, carry):
                    base = pl.multiple_of(s * EPT + i * CH, 8)
                    pltpu.sync_copy(cols.at[pl.ds(base, CH)], colv)
                    pltpu.sync_copy(rows.at[pl.ds(base, CH)], rowv)
                    pltpu.sync_copy(vals.at[pl.ds(base, CH)], valv)
                    pltpu.async_copy(x.at[colv], gath, sem).wait()

                    def scale(w, carry2):
                        vwin = valv[pl.ds(pl.multiple_of(w * 16, 8), 16)]
                        for k2 in range(16):
                            vv = vwin[k2]
                            row = w * 16 + k2
                            for j in range(FB // 16):
                                sl = pl.ds(j * 16, 16)
                                gath[row, sl] = gath[row, sl] * vv
                        return carry2

                    lax.fori_loop(0, CH // 16, scale, 0)
                    pltpu.sync_copy(gath, acc.at[rowv], add=True)
                    return carry

                lax.fori_loop(0, NCHUNK, chunk, 0)

        plsc.subcore_barrier()
        for cc in range(NC):
            blk = p * 2 + cc

            @pl.when((c == cc) & jnp.logical_not(is_last))
            def _(blk=blk):
                pltpu.sync_copy(acc.at[pl.ds(s * RPT, RPT)],
                                outs[blk].at[pl.ds(s * RPT, RPT)])

            @pl.when((c == cc) & is_last)
            def _(blk=blk):
                pltpu.sync_copy(acc.at[pl.ds(15 * RPT, RPT_LAST)],
                                outs[blk].at[pl.ds(15 * RPT, RPT_LAST)])


_spmm = pl.kernel(
    _spmm_body,
    out_type=[jax.ShapeDtypeStruct((N, FB), jnp.float32) for _ in range(NBLK)],
    mesh=plsc.VectorSubcoreMesh(core_axis_name="c", subcore_axis_name="s"),
    scratch_types=[
        pltpu.VMEM((CH,), jnp.int32),
        pltpu.VMEM((CH,), jnp.int32),
        pltpu.VMEM((CH,), jnp.float32),
        pltpu.VMEM((CH, FB), jnp.float32),
        pltpu.VMEM((RPT_LAST, FB), jnp.float32),
        pltpu.VMEM_SHARED((N, FB), jnp.float32),
        pltpu.SemaphoreType.DMA,
    ],
    compiler_params=pltpu.CompilerParams(use_tc_tiling_on_sc=False),
)


_NT = 1000
_S1B = 4
_S2B = 8


def _mix1_body(*refs):
    xrefs = refs[:5 * _S1B]
    a_ref, b_ref, bt_ref, bh_ref = refs[5 * _S1B:5 * _S1B + 4]
    theta_ref = refs[5 * _S1B + 4]
    h_refs = refs[5 * _S1B + 5:]
    xs = [r[...] for r in xrefs]
    acc = bt_ref[...]
    for t in range(5 * _S1B):
        acc = acc + jnp.dot(xs[t], a_ref[t], preferred_element_type=jnp.float32)
    theta_ref[...] = jax.nn.sigmoid(acc)
    for q in range(_S2B):
        accq = bh_ref[:, q * FB:(q + 1) * FB]
        for t in range(5 * _S1B):
            accq = accq + jnp.dot(xs[t], b_ref[t, :, q * FB:(q + 1) * FB],
                                  preferred_element_type=jnp.float32)
        h_refs[q][...] = jnp.tanh(accq)


def _mix2_body(*refs):
    xrefs = refs[:5 * _S2B]
    c_ref, th_ref, bc_ref = refs[5 * _S2B:5 * _S2B + 3]
    out_ref = refs[5 * _S2B + 3]
    acc = bc_ref[...]
    for t in range(5 * _S2B):
        acc = acc + jnp.dot(xrefs[t][...], c_ref[t],
                            preferred_element_type=jnp.float32)
    out_ref[...] = -th_ref[...] * jnp.tanh(acc)


def _blk_spec():
    return pl.BlockSpec((_NT, FB), lambda i: (i, 0))


def _full_spec(shape):
    nd = len(shape)
    return pl.BlockSpec(shape, lambda i, nd=nd: (0,) * nd)


_mix1 = pl.pallas_call(
    _mix1_body,
    grid=(N // _NT,),
    in_specs=[_blk_spec() for _ in range(5 * _S1B)] + [
        _full_spec((5 * _S1B, FB, 256)), _full_spec((5 * _S1B, FB, 512)),
        _full_spec((1, 256)), _full_spec((1, 512)),
    ],
    out_specs=[pl.BlockSpec((_NT, 256), lambda i: (i, 0))] +
              [_blk_spec() for _ in range(_S2B)],
    out_shape=[jax.ShapeDtypeStruct((N, 256), jnp.float32)] +
              [jax.ShapeDtypeStruct((N, FB), jnp.float32) for _ in range(_S2B)],
    compiler_params=pltpu.CompilerParams(dimension_semantics=("arbitrary",)),
)

_mix2 = pl.pallas_call(
    _mix2_body,
    grid=(N // _NT,),
    in_specs=[_blk_spec() for _ in range(5 * _S2B)] + [
        _full_spec((5 * _S2B, FB, 256)),
        pl.BlockSpec((_NT, 256), lambda i: (i, 0)),
        _full_spec((1, 256)),
    ],
    out_specs=pl.BlockSpec((_NT, 256), lambda i: (i, 0)),
    out_shape=jax.ShapeDtypeStruct((N, 256), jnp.float32),
    compiler_params=pltpu.CompilerParams(dimension_semantics=("arbitrary",)),
)


def _spmm8(blocks, r, c, v):
    lo = _spmm(*blocks[:4], r, c, v)
    hi = _spmm(*blocks[4:], r, c, v)
    return list(lo) + list(hi)


def _expand8(w):
    i_dim, o_dim = w.shape
    return jnp.einsum("io,bd->ibod", w, jnp.eye(8, dtype=jnp.float32)).reshape(
        i_dim * 8, o_dim * 8)


def _fold(w, in_dim):
    wr = w.reshape(in_dim, NUM_MAT, w.shape[1])
    w0, w1, w2, w3, w4 = [wr[:, m, :] for m in range(NUM_MAT)]
    mats = [w0 - w2 - w4, w1, 2.0 * w2, w3, 2.0 * w4]
    return jnp.stack([_expand8(m) for m in mats])


def kernel(t_local, y, W_theta, b_lat, W_hid, b_units, W_out,
           s1_rows, s1_cols, s1_vals, s2_rows, s2_cols, s2_vals):
    del t_local
    x0 = y.reshape(B, N, LATENT).transpose(1, 2, 0).reshape(N, _S1B, FB)
    xb = [x0[:, j, :] for j in range(_S1B)]

    z1 = _spmm(*xb, s1_rows, s1_cols, s1_vals)
    z2 = _spmm(*z1, s1_rows, s1_cols, s1_vals)
    z3 = _spmm(*xb, s2_rows, s2_cols, s2_vals)
    z4 = _spmm(*z3, s2_rows, s2_cols, s2_vals)

    a_m = _fold(W_theta, LATENT).reshape(5 * _S1B, FB, 256)
    b_m = _fold(W_hid, LATENT).reshape(5 * _S1B, FB, 512)
    c_m = _fold(W_out, UNITS).reshape(5 * _S2B, FB, 256)
    bt = jnp.repeat(b_lat, 8).reshape(1, 256)
    bh = jnp.repeat(b_units, 8).reshape(1, 512)

    mix1_out = _mix1(*xb, *z1, *z2, *z3, *z4, a_m, b_m, bt, bh)
    theta, hb = mix1_out[0], list(mix1_out[1:])

    w1 = _spmm8(hb, s1_rows, s1_cols, s1_vals)
    w2 = _spmm8(w1, s1_rows, s1_cols, s1_vals)
    w3 = _spmm8(hb, s2_rows, s2_cols, s2_vals)
    w4 = _spmm8(w3, s2_rows, s2_cols, s2_vals)

    out_nb = _mix2(*hb, *w1, *w2, *w3, *w4, c_m, theta, bt)
    return out_nb.reshape(N, LATENT, B).transpose(2, 0, 1).reshape(B, N * LATENT)


# speedup vs baseline: 2.1150x; 2.1150x over previous
import jax
import jax.numpy as jnp
from jax import lax
from jax.experimental import pallas as pl
from jax.experimental.pallas import tpu as pltpu
from jax.experimental.pallas import tpu_sc as plsc

N = 10000
NNZ = 160000
B = 8
LATENT = 32
UNITS = 64
NUM_MAT = 5

NC = 2
NS = 16
FB = 64
NBLK = 4
EPT = NNZ // NS
CH = 80
NCHUNK = EPT // CH
RPT = 624
RPT_LAST = N - 15 * RPT


def _spmm_body(*refs):
    xs = refs[:NBLK]
    rows, cols, vals = refs[NBLK:NBLK + 3]
    outs = refs[NBLK + 3:NBLK + 3 + NBLK]
    (colblk, rowblk, valblk, g0, g1, zbuf, acc,
     sem_i, sem_g0, sem_g1) = refs[NBLK + 3 + NBLK:]
    c = lax.axis_index("c")
    s = lax.axis_index("s")
    zeros16 = jnp.zeros((16,), jnp.float32)
    gbufs = (g0, g1)
    gsems = (sem_g0, sem_g1)

    def zb(i, carry):
        for j in range(FB // 16):
            zbuf[i, pl.ds(j * 16, 16)] = zeros16
        return carry

    lax.fori_loop(0, RPT_LAST, zb, 0)
    is_last = s == NS - 1

    pltpu.async_copy(cols.at[pl.ds(s * NCHUNK, NCHUNK)], colblk, sem_i).wait()
    pltpu.async_copy(rows.at[pl.ds(s * NCHUNK, NCHUNK)], rowblk, sem_i).wait()
    pltpu.async_copy(vals.at[pl.ds(s * NCHUNK, NCHUNK)], valblk, sem_i).wait()

    for p in range(NBLK // 2):
        @pl.when(jnp.logical_not(is_last))
        def _():
            pltpu.sync_copy(zbuf.at[pl.ds(0, RPT)], acc.at[pl.ds(s * RPT, RPT)])

        @pl.when(is_last)
        def _():
            pltpu.sync_copy(zbuf, acc.at[pl.ds(15 * RPT, RPT_LAST)])

        plsc.subcore_barrier()
        for cc in range(NC):
            blk = p * 2 + cc

            @pl.when(c == cc)
            def _(blk=blk):
                x = xs[blk]

                def start_gather(j, b):
                    pltpu.async_copy(x.at[colblk.at[j]], gbufs[b], gsems[b])

                def process(j, b):
                    pltpu.make_async_copy(x.at[colblk.at[j]], gbufs[b],
                                          gsems[b]).wait()
                    gath = gbufs[b]

                    def scale(w, carry2):
                        vwin = valblk[j, pl.ds(pl.multiple_of(w * 16, 8), 16)]
                        for k2 in range(16):
                            vv = vwin[k2]
                            row = w * 16 + k2
                            for jj in range(FB // 16):
                                sl = pl.ds(jj * 16, 16)
                                gath[row, sl] = gath[row, sl] * vv
                        return carry2

                    lax.fori_loop(0, CH // 16, scale, 0)
                    pltpu.sync_copy(gath, acc.at[rowblk.at[j]], add=True)

                start_gather(0, 0)
                start_gather(1, 1)

                def chunk(i, carry):
                    j = i * 2
                    process(j, 0)

                    @pl.when(j + 2 < NCHUNK)
                    def _():
                        start_gather(j + 2, 0)

                    process(j + 1, 1)

                    @pl.when(j + 3 < NCHUNK)
                    def _():
                        start_gather(j + 3, 1)

                    return carry

                lax.fori_loop(0, NCHUNK // 2, chunk, 0)
                process(NCHUNK - 1, 0)

        plsc.subcore_barrier()
        for cc in range(NC):
            blk = p * 2 + cc

            @pl.when((c == cc) & jnp.logical_not(is_last))
            def _(blk=blk):
                pltpu.sync_copy(acc.at[pl.ds(s * RPT, RPT)],
                                outs[blk].at[pl.ds(s * RPT, RPT)])

            @pl.when((c == cc) & is_last)
            def _(blk=blk):
                pltpu.sync_copy(acc.at[pl.ds(15 * RPT, RPT_LAST)],
                                outs[blk].at[pl.ds(15 * RPT, RPT_LAST)])


_spmm = pl.kernel(
    _spmm_body,
    out_type=[jax.ShapeDtypeStruct((N, FB), jnp.float32) for _ in range(NBLK)],
    mesh=plsc.VectorSubcoreMesh(core_axis_name="c", subcore_axis_name="s"),
    scratch_types=[
        pltpu.VMEM((NCHUNK, CH), jnp.int32),
        pltpu.VMEM((NCHUNK, CH), jnp.int32),
        pltpu.VMEM((NCHUNK, CH), jnp.float32),
        pltpu.VMEM((CH, FB), jnp.float32),
        pltpu.VMEM((CH, FB), jnp.float32),
        pltpu.VMEM((RPT_LAST, FB), jnp.float32),
        pltpu.VMEM_SHARED((N, FB), jnp.float32),
        pltpu.SemaphoreType.DMA,
        pltpu.SemaphoreType.DMA,
        pltpu.SemaphoreType.DMA,
    ],
    compiler_params=pltpu.CompilerParams(use_tc_tiling_on_sc=False),
)


_NT = 1000
_S1B = 4
_S2B = 8


def _mix1_body(*refs):
    xrefs = refs[:5 * _S1B]
    a_ref, b_ref, bt_ref, bh_ref = refs[5 * _S1B:5 * _S1B + 4]
    theta_ref = refs[5 * _S1B + 4]
    h_refs = refs[5 * _S1B + 5:]
    xs = [r[...] for r in xrefs]
    acc = bt_ref[...]
    for t in range(5 * _S1B):
        acc = acc + jnp.dot(xs[t], a_ref[t], preferred_element_type=jnp.float32)
    theta_ref[...] = jax.nn.sigmoid(acc)
    for q in range(_S2B):
        accq = bh_ref[:, q * FB:(q + 1) * FB]
        for t in range(5 * _S1B):
            accq = accq + jnp.dot(xs[t], b_ref[t, :, q * FB:(q + 1) * FB],
                                  preferred_element_type=jnp.float32)
        h_refs[q][...] = jnp.tanh(accq)


def _mix2_body(*refs):
    xrefs = refs[:5 * _S2B]
    c_ref, th_ref, bc_ref = refs[5 * _S2B:5 * _S2B + 3]
    out_ref = refs[5 * _S2B + 3]
    acc = bc_ref[...]
    for t in range(5 * _S2B):
        acc = acc + jnp.dot(xrefs[t][...], c_ref[t],
                            preferred_element_type=jnp.float32)
    out_ref[...] = -th_ref[...] * jnp.tanh(acc)


def _blk_spec():
    return pl.BlockSpec((_NT, FB), lambda i: (i, 0))


def _full_spec(shape):
    nd = len(shape)
    return pl.BlockSpec(shape, lambda i, nd=nd: (0,) * nd)


_mix1 = pl.pallas_call(
    _mix1_body,
    grid=(N // _NT,),
    in_specs=[_blk_spec() for _ in range(5 * _S1B)] + [
        _full_spec((5 * _S1B, FB, 256)), _full_spec((5 * _S1B, FB, 512)),
        _full_spec((1, 256)), _full_spec((1, 512)),
    ],
    out_specs=[pl.BlockSpec((_NT, 256), lambda i: (i, 0))] +
              [_blk_spec() for _ in range(_S2B)],
    out_shape=[jax.ShapeDtypeStruct((N, 256), jnp.float32)] +
              [jax.ShapeDtypeStruct((N, FB), jnp.float32) for _ in range(_S2B)],
    compiler_params=pltpu.CompilerParams(dimension_semantics=("arbitrary",)),
)

_mix2 = pl.pallas_call(
    _mix2_body,
    grid=(N // _NT,),
    in_specs=[_blk_spec() for _ in range(5 * _S2B)] + [
        _full_spec((5 * _S2B, FB, 256)),
        pl.BlockSpec((_NT, 256), lambda i: (i, 0)),
        _full_spec((1, 256)),
    ],
    out_specs=pl.BlockSpec((_NT, 256), lambda i: (i, 0)),
    out_shape=jax.ShapeDtypeStruct((N, 256), jnp.float32),
    compiler_params=pltpu.CompilerParams(dimension_semantics=("arbitrary",)),
)


def _spmm8(blocks, r, c, v):
    lo = _spmm(*blocks[:4], r, c, v)
    hi = _spmm(*blocks[4:], r, c, v)
    return list(lo) + list(hi)


def _expand8(w):
    i_dim, o_dim = w.shape
    return jnp.einsum("io,bd->ibod", w, jnp.eye(8, dtype=jnp.float32)).reshape(
        i_dim * 8, o_dim * 8)


def _fold(w, in_dim):
    wr = w.reshape(in_dim, NUM_MAT, w.shape[1])
    w0, w1, w2, w3, w4 = [wr[:, m, :] for m in range(NUM_MAT)]
    mats = [w0 - w2 - w4, w1, 2.0 * w2, w3, 2.0 * w4]
    return jnp.stack([_expand8(m) for m in mats])


def kernel(t_local, y, W_theta, b_lat, W_hid, b_units, W_out,
           s1_rows, s1_cols, s1_vals, s2_rows, s2_cols, s2_vals):
    del t_local
    s1_rows = s1_rows.reshape(NS * NCHUNK, CH)
    s1_cols = s1_cols.reshape(NS * NCHUNK, CH)
    s1_vals = s1_vals.reshape(NS * NCHUNK, CH)
    s2_rows = s2_rows.reshape(NS * NCHUNK, CH)
    s2_cols = s2_cols.reshape(NS * NCHUNK, CH)
    s2_vals = s2_vals.reshape(NS * NCHUNK, CH)
    x0 = y.reshape(B, N, LATENT).transpose(1, 2, 0).reshape(N, _S1B, FB)
    xb = [x0[:, j, :] for j in range(_S1B)]

    z1 = _spmm(*xb, s1_rows, s1_cols, s1_vals)
    z2 = _spmm(*z1, s1_rows, s1_cols, s1_vals)
    z3 = _spmm(*xb, s2_rows, s2_cols, s2_vals)
    z4 = _spmm(*z3, s2_rows, s2_cols, s2_vals)

    a_m = _fold(W_theta, LATENT).reshape(5 * _S1B, FB, 256)
    b_m = _fold(W_hid, LATENT).reshape(5 * _S1B, FB, 512)
    c_m = _fold(W_out, UNITS).reshape(5 * _S2B, FB, 256)
    bt = jnp.repeat(b_lat, 8).reshape(1, 256)
    bh = jnp.repeat(b_units, 8).reshape(1, 512)

    mix1_out = _mix1(*xb, *z1, *z2, *z3, *z4, a_m, b_m, bt, bh)
    theta, hb = mix1_out[0], list(mix1_out[1:])

    w1 = _spmm8(hb, s1_rows, s1_cols, s1_vals)
    w2 = _spmm8(w1, s1_rows, s1_cols, s1_vals)
    w3 = _spmm8(hb, s2_rows, s2_cols, s2_vals)
    w4 = _spmm8(w3, s2_rows, s2_cols, s2_vals)

    out_nb = _mix2(*hb, *w1, *w2, *w3, *w4, c_m, theta, bt)
    return out_nb.reshape(N, LATENT, B).transpose(2, 0, 1).reshape(B, N * LATENT)


# speedup vs baseline: 4.0422x; 1.9112x over previous
import jax
import jax.numpy as jnp
from jax import lax
from jax.experimental import pallas as pl
from jax.experimental.pallas import tpu as pltpu
from jax.experimental.pallas import tpu_sc as plsc

N = 10000
NNZ = 160000
B = 8
LATENT = 32
UNITS = 64
NUM_MAT = 5

NC = 2
NS = 16
FB = 64
NBLK = 4
EPT = NNZ // NS
CH = 80
NCHUNK = EPT // CH
RPT = 624
RPT_LAST = N - 15 * RPT


def _spmm_body(*refs):
    xs = refs[:NBLK]
    rows, cols, vals = refs[NBLK:NBLK + 3]
    outs = refs[NBLK + 3:NBLK + 3 + NBLK]
    (colblk, rowblk, valblk, g0, g1, scl, zbuf, acc,
     sem_i, sem_g0, sem_g1) = refs[NBLK + 3 + NBLK:]
    c = lax.axis_index("c")
    s = lax.axis_index("s")
    zeros16 = jnp.zeros((16,), jnp.float32)
    gbufs = (g0, g1)
    gsems = (sem_g0, sem_g1)

    def zb(i, carry):
        for j in range(FB // 16):
            zbuf[i, pl.ds(j * 16, 16)] = zeros16
        return carry

    lax.fori_loop(0, RPT_LAST, zb, 0)
    is_last = s == NS - 1

    pltpu.async_copy(cols.at[pl.ds(s * NCHUNK, NCHUNK)], colblk, sem_i).wait()
    pltpu.async_copy(rows.at[pl.ds(s * NCHUNK, NCHUNK)], rowblk, sem_i).wait()
    pltpu.async_copy(vals.at[pl.ds(s * NCHUNK, NCHUNK)], valblk, sem_i).wait()

    for p in range(NBLK // 2):
        @pl.when(jnp.logical_not(is_last))
        def _():
            pltpu.sync_copy(zbuf.at[pl.ds(0, RPT)], acc.at[pl.ds(s * RPT, RPT)])

        @pl.when(is_last)
        def _():
            pltpu.sync_copy(zbuf, acc.at[pl.ds(15 * RPT, RPT_LAST)])

        plsc.subcore_barrier()
        for cc in range(NC):
            blk = p * 2 + cc

            @pl.when(c == cc)
            def _(blk=blk):
                x = xs[blk]

                def start_gather(j, b):
                    pltpu.async_copy(x.at[colblk.at[j]], gbufs[b], gsems[b])

                def process(j, b):
                    pltpu.make_async_copy(x.at[colblk.at[j]], gbufs[b],
                                          gsems[b]).wait()
                    gath = gbufs[b]

                    def scale(w, carry2):
                        vwin = valblk[j, pl.ds(pl.multiple_of(w * 16, 8), 16)]
                        for k2 in range(16):
                            vv = vwin[k2]
                            row = w * 16 + k2
                            for jj in range(FB // 16):
                                sl = pl.ds(jj * 16, 16)
                                scl[row, sl] = gath[row, sl] * vv
                        return carry2

                    lax.fori_loop(0, CH // 16, scale, 0)
                    pltpu.sync_copy(scl, acc.at[rowblk.at[j]], add=True)

                start_gather(0, 0)
                start_gather(1, 1)

                def chunk(i, carry):
                    j = i * 2
                    process(j, 0)

                    @pl.when(j + 2 < NCHUNK)
                    def _():
                        start_gather(j + 2, 0)

                    process(j + 1, 1)

                    @pl.when(j + 3 < NCHUNK)
                    def _():
                        start_gather(j + 3, 1)

                    return carry

                lax.fori_loop(0, NCHUNK // 2, chunk, 0)
                process(NCHUNK - 1, 0)

        plsc.subcore_barrier()
        for cc in range(NC):
            blk = p * 2 + cc

            @pl.when((c == cc) & jnp.logical_not(is_last))
            def _(blk=blk):
                pltpu.sync_copy(acc.at[pl.ds(s * RPT, RPT)],
                                outs[blk].at[pl.ds(s * RPT, RPT)])

            @pl.when((c == cc) & is_last)
            def _(blk=blk):
                pltpu.sync_copy(acc.at[pl.ds(15 * RPT, RPT_LAST)],
                                outs[blk].at[pl.ds(15 * RPT, RPT_LAST)])


_spmm = pl.kernel(
    _spmm_body,
    out_type=[jax.ShapeDtypeStruct((N, FB), jnp.float32) for _ in range(NBLK)],
    mesh=plsc.VectorSubcoreMesh(core_axis_name="c", subcore_axis_name="s"),
    scratch_types=[
        pltpu.VMEM((NCHUNK, CH), jnp.int32),
        pltpu.VMEM((NCHUNK, CH), jnp.int32),
        pltpu.VMEM((NCHUNK, CH), jnp.float32),
        pltpu.VMEM((CH, FB), jnp.float32),
        pltpu.VMEM((CH, FB), jnp.float32),
        pltpu.VMEM((CH, FB), jnp.float32),
        pltpu.VMEM((RPT_LAST, FB), jnp.float32),
        pltpu.VMEM_SHARED((N, FB), jnp.float32),
        pltpu.SemaphoreType.DMA,
        pltpu.SemaphoreType.DMA,
        pltpu.SemaphoreType.DMA,
    ],
    compiler_params=pltpu.CompilerParams(use_tc_tiling_on_sc=False),
)


_NT = 1000
_S1B = 4
_S2B = 8


def _mix1_body(*refs):
    xrefs = refs[:5 * _S1B]
    a_ref, b_ref, bt_ref, bh_ref = refs[5 * _S1B:5 * _S1B + 4]
    theta_ref = refs[5 * _S1B + 4]
    h_refs = refs[5 * _S1B + 5:]
    xs = [r[...] for r in xrefs]
    acc = bt_ref[...]
    for t in range(5 * _S1B):
        acc = acc + jnp.dot(xs[t], a_ref[t], preferred_element_type=jnp.float32)
    theta_ref[...] = jax.nn.sigmoid(acc)
    for q in range(_S2B):
        accq = bh_ref[:, q * FB:(q + 1) * FB]
        for t in range(5 * _S1B):
            accq = accq + jnp.dot(xs[t], b_ref[t, :, q * FB:(q + 1) * FB],
                                  preferred_element_type=jnp.float32)
        h_refs[q][...] = jnp.tanh(accq)


def _mix2_body(*refs):
    xrefs = refs[:5 * _S2B]
    c_ref, th_ref, bc_ref = refs[5 * _S2B:5 * _S2B + 3]
    out_ref = refs[5 * _S2B + 3]
    acc = bc_ref[...]
    for t in range(5 * _S2B):
        acc = acc + jnp.dot(xrefs[t][...], c_ref[t],
                            preferred_element_type=jnp.float32)
    out_ref[...] = -th_ref[...] * jnp.tanh(acc)


def _blk_spec():
    return pl.BlockSpec((_NT, FB), lambda i: (i, 0))


def _full_spec(shape):
    nd = len(shape)
    return pl.BlockSpec(shape, lambda i, nd=nd: (0,) * nd)


_mix1 = pl.pallas_call(
    _mix1_body,
    grid=(N // _NT,),
    in_specs=[_blk_spec() for _ in range(5 * _S1B)] + [
        _full_spec((5 * _S1B, FB, 256)), _full_spec((5 * _S1B, FB, 512)),
        _full_spec((1, 256)), _full_spec((1, 512)),
    ],
    out_specs=[pl.BlockSpec((_NT, 256), lambda i: (i, 0))] +
              [_blk_spec() for _ in range(_S2B)],
    out_shape=[jax.ShapeDtypeStruct((N, 256), jnp.float32)] +
              [jax.ShapeDtypeStruct((N, FB), jnp.float32) for _ in range(_S2B)],
    compiler_params=pltpu.CompilerParams(dimension_semantics=("arbitrary",)),
)

_mix2 = pl.pallas_call(
    _mix2_body,
    grid=(N // _NT,),
    in_specs=[_blk_spec() for _ in range(5 * _S2B)] + [
        _full_spec((5 * _S2B, FB, 256)),
        pl.BlockSpec((_NT, 256), lambda i: (i, 0)),
        _full_spec((1, 256)),
    ],
    out_specs=pl.BlockSpec((_NT, 256), lambda i: (i, 0)),
    out_shape=jax.ShapeDtypeStruct((N, 256), jnp.float32),
    compiler_params=pltpu.CompilerParams(dimension_semantics=("arbitrary",)),
)


def _spmm8(blocks, r, c, v):
    lo = _spmm(*blocks[:4], r, c, v)
    hi = _spmm(*blocks[4:], r, c, v)
    return list(lo) + list(hi)


def _expand8(w):
    i_dim, o_dim = w.shape
    return jnp.einsum("io,bd->ibod", w, jnp.eye(8, dtype=jnp.float32)).reshape(
        i_dim * 8, o_dim * 8)


def _fold(w, in_dim):
    wr = w.reshape(in_dim, NUM_MAT, w.shape[1])
    w0, w1, w2, w3, w4 = [wr[:, m, :] for m in range(NUM_MAT)]
    mats = [w0 - w2 - w4, w1, 2.0 * w2, w3, 2.0 * w4]
    return jnp.stack([_expand8(m) for m in mats])


def kernel(t_local, y, W_theta, b_lat, W_hid, b_units, W_out,
           s1_rows, s1_cols, s1_vals, s2_rows, s2_cols, s2_vals):
    del t_local
    s1_rows = s1_rows.reshape(NS * NCHUNK, CH)
    s1_cols = s1_cols.reshape(NS * NCHUNK, CH)
    s1_vals = s1_vals.reshape(NS * NCHUNK, CH)
    s2_rows = s2_rows.reshape(NS * NCHUNK, CH)
    s2_cols = s2_cols.reshape(NS * NCHUNK, CH)
    s2_vals = s2_vals.reshape(NS * NCHUNK, CH)
    x0 = y.reshape(B, N, LATENT).transpose(1, 2, 0).reshape(N, _S1B, FB)
    xb = [x0[:, j, :] for j in range(_S1B)]

    z1 = _spmm(*xb, s1_rows, s1_cols, s1_vals)
    z2 = _spmm(*z1, s1_rows, s1_cols, s1_vals)
    z3 = _spmm(*xb, s2_rows, s2_cols, s2_vals)
    z4 = _spmm(*z3, s2_rows, s2_cols, s2_vals)

    a_m = _fold(W_theta, LATENT).reshape(5 * _S1B, FB, 256)
    b_m = _fold(W_hid, LATENT).reshape(5 * _S1B, FB, 512)
    c_m = _fold(W_out, UNITS).reshape(5 * _S2B, FB, 256)
    bt = jnp.repeat(b_lat, 8).reshape(1, 256)
    bh = jnp.repeat(b_units, 8).reshape(1, 512)

    mix1_out = _mix1(*xb, *z1, *z2, *z3, *z4, a_m, b_m, bt, bh)
    theta, hb = mix1_out[0], list(mix1_out[1:])

    w1 = _spmm8(hb, s1_rows, s1_cols, s1_vals)
    w2 = _spmm8(w1, s1_rows, s1_cols, s1_vals)
    w3 = _spmm8(hb, s2_rows, s2_cols, s2_vals)
    w4 = _spmm8(w3, s2_rows, s2_cols, s2_vals)

    out_nb = _mix2(*hb, *w1, *w2, *w3, *w4, c_m, theta, bt)
    return out_nb.reshape(N, LATENT, B).transpose(2, 0, 1).reshape(B, N * LATENT)


# speedup vs baseline: 4.1362x; 1.0233x over previous
import jax
import jax.numpy as jnp
from jax import lax
from jax.experimental import pallas as pl
from jax.experimental.pallas import tpu as pltpu
from jax.experimental.pallas import tpu_sc as plsc

N = 10000
NNZ = 160000
B = 8
LATENT = 32
UNITS = 64
NUM_MAT = 5

NC = 2
NS = 16
FB = 64
NBLK = 4
EPT = NNZ // NS
CH = 80
NCHUNK = EPT // CH
RPT = 624
RPT_LAST = N - 15 * RPT
ZR = 16


def _spmm_body(*refs):
    xs = refs[:NBLK]
    rows, cols, vals = refs[NBLK:NBLK + 3]
    outs = refs[NBLK + 3:NBLK + 3 + NBLK]
    (colblk, rowblk, valblk, g0, g1, scl0, scl1, zbuf, acc,
     sem_i, sem_g0, sem_g1, sem_s0, sem_s1) = refs[NBLK + 3 + NBLK:]
    c = lax.axis_index("c")
    s = lax.axis_index("s")
    zeros16 = jnp.zeros((16,), jnp.float32)
    gbufs = (g0, g1)
    gsems = (sem_g0, sem_g1)
    sbufs = (scl0, scl1)
    ssems = (sem_s0, sem_s1)

    def zb(i, carry):
        for j in range(FB // 16):
            zbuf[i, pl.ds(j * 16, 16)] = zeros16
        return carry

    lax.fori_loop(0, ZR, zb, 0)
    is_last = s == NS - 1

    pltpu.async_copy(cols.at[pl.ds(s * NCHUNK, NCHUNK)], colblk, sem_i).wait()
    pltpu.async_copy(rows.at[pl.ds(s * NCHUNK, NCHUNK)], rowblk, sem_i).wait()
    pltpu.async_copy(vals.at[pl.ds(s * NCHUNK, NCHUNK)], valblk, sem_i).wait()

    for p in range(NBLK // 2):
        for t in range(RPT // ZR):
            pltpu.async_copy(zbuf, acc.at[pl.ds(s * RPT + t * ZR, ZR)], sem_i)

        @pl.when(is_last)
        def _():
            pltpu.async_copy(zbuf, acc.at[pl.ds(15 * RPT + RPT, ZR)], sem_i)

        for t in range(RPT // ZR):
            pltpu.make_async_copy(zbuf, acc.at[pl.ds(0, ZR)], sem_i).wait()

        @pl.when(is_last)
        def _():
            pltpu.make_async_copy(zbuf, acc.at[pl.ds(0, ZR)], sem_i).wait()

        plsc.subcore_barrier()
        for cc in range(NC):
            blk = p * 2 + cc

            @pl.when(c == cc)
            def _(blk=blk):
                x = xs[blk]

                def start_gather(j, b):
                    pltpu.async_copy(x.at[colblk.at[j]], gbufs[b], gsems[b])

                def wait_scatter(b):
                    pltpu.make_async_copy(sbufs[b], acc.at[rowblk.at[0]],
                                          ssems[b]).wait()

                def process(j, b, first):
                    pltpu.make_async_copy(x.at[colblk.at[j]], gbufs[b],
                                          gsems[b]).wait()
                    if not first:
                        wait_scatter(b)
                    gath = gbufs[b]
                    scl = sbufs[b]

                    def scale(w, carry2):
                        vwin = valblk[j, pl.ds(pl.multiple_of(w * 16, 8), 16)]
                        for k2 in range(16):
                            vv = vwin[k2]
                            row = w * 16 + k2
                            for jj in range(FB // 16):
                                sl = pl.ds(jj * 16, 16)
                                scl[row, sl] = gath[row, sl] * vv
                        return carry2

                    lax.fori_loop(0, CH // 16, scale, 0)
                    pltpu.async_copy(scl, acc.at[rowblk.at[j]], ssems[b],
                                     add=True)

                start_gather(0, 0)
                start_gather(1, 1)
                process(0, 0, True)
                start_gather(2, 0)
                process(1, 1, True)
                start_gather(3, 1)

                def chunk(i, carry):
                    j = i * 2 + 2
                    process(j, 0, False)

                    @pl.when(j + 2 < NCHUNK)
                    def _():
                        start_gather(j + 2, 0)

                    process(j + 1, 1, False)

                    @pl.when(j + 3 < NCHUNK)
                    def _():
                        start_gather(j + 3, 1)

                    return carry

                lax.fori_loop(0, (NCHUNK - 3) // 2, chunk, 0)
                process(NCHUNK - 1, 0, False)
                wait_scatter(0)
                wait_scatter(1)

        plsc.subcore_barrier()
        for cc in range(NC):
            blk = p * 2 + cc

            @pl.when((c == cc) & (s < NS - 1))
            def _(blk=blk):
                pltpu.sync_copy(acc.at[pl.ds(s * RPT, RPT)],
                                outs[blk].at[pl.ds(s * RPT, RPT)])

            @pl.when((c == cc) & is_last)
            def _(blk=blk):
                pltpu.sync_copy(acc.at[pl.ds(15 * RPT, RPT_LAST)],
                                outs[blk].at[pl.ds(15 * RPT, RPT_LAST)])


_spmm = pl.kernel(
    _spmm_body,
    out_type=[jax.ShapeDtypeStruct((N, FB), jnp.float32) for _ in range(NBLK)],
    mesh=plsc.VectorSubcoreMesh(core_axis_name="c", subcore_axis_name="s"),
    scratch_types=[
        pltpu.VMEM((NCHUNK, CH), jnp.int32),
        pltpu.VMEM((NCHUNK, CH), jnp.int32),
        pltpu.VMEM((NCHUNK, CH), jnp.float32),
        pltpu.VMEM((CH, FB), jnp.float32),
        pltpu.VMEM((CH, FB), jnp.float32),
        pltpu.VMEM((CH, FB), jnp.float32),
        pltpu.VMEM((CH, FB), jnp.float32),
        pltpu.VMEM((ZR, FB), jnp.float32),
        pltpu.VMEM_SHARED((N, FB), jnp.float32),
        pltpu.SemaphoreType.DMA,
        pltpu.SemaphoreType.DMA,
        pltpu.SemaphoreType.DMA,
        pltpu.SemaphoreType.DMA,
        pltpu.SemaphoreType.DMA,
    ],
    compiler_params=pltpu.CompilerParams(use_tc_tiling_on_sc=False),
)


_NT = 1000
_S1B = 4
_S2B = 8


def _mix1_body(*refs):
    xrefs = refs[:5 * _S1B]
    a_ref, b_ref, bt_ref, bh_ref = refs[5 * _S1B:5 * _S1B + 4]
    theta_ref = refs[5 * _S1B + 4]
    h_refs = refs[5 * _S1B + 5:]
    xs = [r[...] for r in xrefs]
    acc = bt_ref[...]
    for t in range(5 * _S1B):
        acc = acc + jnp.dot(xs[t], a_ref[t], preferred_element_type=jnp.float32)
    theta_ref[...] = jax.nn.sigmoid(acc)
    for q in range(_S2B):
        accq = bh_ref[:, q * FB:(q + 1) * FB]
        for t in range(5 * _S1B):
            accq = accq + jnp.dot(xs[t], b_ref[t, :, q * FB:(q + 1) * FB],
                                  preferred_element_type=jnp.float32)
        h_refs[q][...] = jnp.tanh(accq)


def _mix2_body(*refs):
    xrefs = refs[:5 * _S2B]
    c_ref, th_ref, bc_ref = refs[5 * _S2B:5 * _S2B + 3]
    out_ref = refs[5 * _S2B + 3]
    acc = bc_ref[...]
    for t in range(5 * _S2B):
        acc = acc + jnp.dot(xrefs[t][...], c_ref[t],
                            preferred_element_type=jnp.float32)
    out_ref[...] = -th_ref[...] * jnp.tanh(acc)


def _blk_spec():
    return pl.BlockSpec((_NT, FB), lambda i: (i, 0))


def _full_spec(shape):
    nd = len(shape)
    return pl.BlockSpec(shape, lambda i, nd=nd: (0,) * nd)


_mix1 = pl.pallas_call(
    _mix1_body,
    grid=(N // _NT,),
    in_specs=[_blk_spec() for _ in range(5 * _S1B)] + [
        _full_spec((5 * _S1B, FB, 256)), _full_spec((5 * _S1B, FB, 512)),
        _full_spec((1, 256)), _full_spec((1, 512)),
    ],
    out_specs=[pl.BlockSpec((_NT, 256), lambda i: (i, 0))] +
              [_blk_spec() for _ in range(_S2B)],
    out_shape=[jax.ShapeDtypeStruct((N, 256), jnp.float32)] +
              [jax.ShapeDtypeStruct((N, FB), jnp.float32) for _ in range(_S2B)],
    compiler_params=pltpu.CompilerParams(dimension_semantics=("arbitrary",)),
)

_mix2 = pl.pallas_call(
    _mix2_body,
    grid=(N // _NT,),
    in_specs=[_blk_spec() for _ in range(5 * _S2B)] + [
        _full_spec((5 * _S2B, FB, 256)),
        pl.BlockSpec((_NT, 256), lambda i: (i, 0)),
        _full_spec((1, 256)),
    ],
    out_specs=pl.BlockSpec((_NT, 256), lambda i: (i, 0)),
    out_shape=jax.ShapeDtypeStruct((N, 256), jnp.float32),
    compiler_params=pltpu.CompilerParams(dimension_semantics=("arbitrary",)),
)


def _gate(xs, deps):
    return lax.optimization_barrier((tuple(xs), tuple(deps)))[0]


def _expand8(w):
    i_dim, o_dim = w.shape
    return jnp.einsum("io,bd->ibod", w, jnp.eye(8, dtype=jnp.float32)).reshape(
        i_dim * 8, o_dim * 8)


def _fold(w, in_dim):
    wr = w.reshape(in_dim, NUM_MAT, w.shape[1])
    w0, w1, w2, w3, w4 = [wr[:, m, :] for m in range(NUM_MAT)]
    mats = [w0 - w2 - w4, w1, 2.0 * w2, w3, 2.0 * w4]
    return jnp.stack([_expand8(m) for m in mats])


def kernel(t_local, y, W_theta, b_lat, W_hid, b_units, W_out,
           s1_rows, s1_cols, s1_vals, s2_rows, s2_cols, s2_vals):
    del t_local
    s1_rows = s1_rows.reshape(NS * NCHUNK, CH)
    s1_cols = s1_cols.reshape(NS * NCHUNK, CH)
    s1_vals = s1_vals.reshape(NS * NCHUNK, CH)
    s2_rows = s2_rows.reshape(NS * NCHUNK, CH)
    s2_cols = s2_cols.reshape(NS * NCHUNK, CH)
    s2_vals = s2_vals.reshape(NS * NCHUNK, CH)
    x0 = y.reshape(B, N, LATENT).transpose(1, 2, 0).reshape(N, _S1B, FB)
    xb = [x0[:, j, :] for j in range(_S1B)]

    z1 = _spmm(*xb, s1_rows, s1_cols, s1_vals)
    z3 = _spmm(*_gate(xb, z1), s2_rows, s2_cols, s2_vals)
    z2 = _spmm(*_gate(z1, z3), s1_rows, s1_cols, s1_vals)
    z4 = _spmm(*_gate(z3, z2), s2_rows, s2_cols, s2_vals)

    a_m = _fold(W_theta, LATENT).reshape(5 * _S1B, FB, 256)
    b_m = _fold(W_hid, LATENT).reshape(5 * _S1B, FB, 512)
    c_m = _fold(W_out, UNITS).reshape(5 * _S2B, FB, 256)
    bt = jnp.repeat(b_lat, 8).reshape(1, 256)
    bh = jnp.repeat(b_units, 8).reshape(1, 512)

    mix1_out = _mix1(*xb, *z1, *z2, *z3, *z4, a_m, b_m, bt, bh)
    theta, hb = mix1_out[0], list(mix1_out[1:])

    w1a = _spmm(*_gate(hb[:4], z4), s1_rows, s1_cols, s1_vals)
    w1b = _spmm(*_gate(hb[4:], w1a), s1_rows, s1_cols, s1_vals)
    w1 = list(w1a) + list(w1b)
    w3a = _spmm(*_gate(hb[:4], w1b), s2_rows, s2_cols, s2_vals)
    w3b = _spmm(*_gate(hb[4:], w3a), s2_rows, s2_cols, s2_vals)
    w3 = list(w3a) + list(w3b)
    w2a = _spmm(*_gate(w1[:4], w3b), s1_rows, s1_cols, s1_vals)
    w2b = _spmm(*_gate(w1[4:], w2a), s1_rows, s1_cols, s1_vals)
    w2 = list(w2a) + list(w2b)
    w4a = _spmm(*_gate(w3[:4], w2b), s2_rows, s2_cols, s2_vals)
    w4b = _spmm(*_gate(w3[4:], w4a), s2_rows, s2_cols, s2_vals)
    w4 = list(w4a) + list(w4b)

    out_nb = _mix2(*hb, *w1, *w2, *w3, *w4, c_m, theta, bt)
    return out_nb.reshape(N, LATENT, B).transpose(2, 0, 1).reshape(B, N * LATENT)
